# Initial kernel scaffold; baseline (speedup 1.0000x reference)
import jax
import jax.numpy as jnp
from jax.experimental import pallas as pl


def kernel(inputs, labels, sample_ids, weight):
    raise NotImplementedError("write your pallas kernel here")


# speedup vs baseline: 1.6524x; 1.6524x over previous
import functools

import jax
import jax.numpy as jnp
from jax import lax
from jax.experimental import pallas as pl
from jax.experimental.pallas import tpu as pltpu
from jax.experimental.pallas import tpu_sc as plsc

_B = 4096
_S = 8192
_D = 128
_BT = 512

_info = plsc.get_sparse_core_info()
_NC = _info.num_cores
_NS = _info.num_subcores
_NW = _NC * _NS
_SPW = _S // _NW
_BPW = _B // _NW


@functools.partial(
    pl.kernel,
    mesh=plsc.VectorSubcoreMesh(core_axis_name="c", subcore_axis_name="s"),
    out_type=(
        jax.ShapeDtypeStruct((_S, _D), jnp.float32),
        jax.ShapeDtypeStruct((_B, _D), jnp.float32),
    ),
    scratch_types=[
        pltpu.VMEM((_SPW,), jnp.int32),
        pltpu.VMEM((_SPW, _D), jnp.float32),
        pltpu.VMEM((_BPW,), jnp.int32),
        pltpu.VMEM((_BPW, _D), jnp.float32),
        pltpu.SemaphoreType.DMA,
        pltpu.SemaphoreType.DMA,
    ],
)
def _sc_gather(weight_hbm, sids_hbm, labels_hbm, out_s, out_t,
               sidx_v, srows_v, lidx_v, lrows_v, sem_s, sem_l):
    wid = lax.axis_index("s") * _NC + lax.axis_index("c")
    sbase = wid * _SPW
    lbase = wid * _BPW
    pltpu.sync_copy(sids_hbm.at[pl.ds(sbase, _SPW)], sidx_v)
    cp_s = pltpu.async_copy(weight_hbm.at[sidx_v], srows_v, sem_s)
    pltpu.sync_copy(labels_hbm.at[pl.ds(lbase, _BPW)], lidx_v)
    cp_l = pltpu.async_copy(weight_hbm.at[lidx_v], lrows_v, sem_l)
    cp_s.wait()
    pltpu.sync_copy(srows_v, out_s.at[pl.ds(sbase, _SPW)])
    cp_l.wait()
    pltpu.sync_copy(lrows_v, out_t.at[pl.ds(lbase, _BPW)])


def _loss_body(x_ref, sw_ref, tw_ref, out_ref):
    i = pl.program_id(0)
    x = x_ref[...]
    logits = lax.dot_general(
        x, sw_ref[...], (((1,), (1,)), ((), ())),
        preferred_element_type=jnp.float32)
    rowsum = jnp.sum(jnp.exp(logits), axis=1)
    true_dot = jnp.sum(x * tw_ref[...], axis=1)
    contrib = jnp.sum(jnp.log(rowsum) - true_dot)

    @pl.when(i == 0)
    def _():
        out_ref[0, 0] = contrib

    @pl.when(i != 0)
    def _():
        out_ref[0, 0] += contrib


def _tc_loss(x, sw, tw):
    out = pl.pallas_call(
        _loss_body,
        grid=(_B // _BT,),
        in_specs=[
            pl.BlockSpec((_BT, _D), lambda i: (i, 0)),
            pl.BlockSpec((_S, _D), lambda i: (0, 0)),
            pl.BlockSpec((_BT, _D), lambda i: (i, 0)),
        ],
        out_specs=pl.BlockSpec((1, 1), lambda i: (0, 0),
                               memory_space=pltpu.SMEM),
        out_shape=jax.ShapeDtypeStruct((1, 1), jnp.float32),
    )(x, sw, tw)
    return out[0, 0]


def kernel(inputs, labels, sample_ids, weight):
    sw, tw = _sc_gather(weight,
                        sample_ids.astype(jnp.int32),
                        labels.astype(jnp.int32))
    return _tc_loss(inputs, sw, tw)
